# baseline (speedup 1.0000x reference)
import functools

import jax
import jax.numpy as jnp
from jax import lax
from jax.experimental import pallas as pl
from jax.experimental.pallas import tpu as pltpu
from jax.experimental.pallas import tpu_sc as plsc

N = 10000
E = 320000
IN_DIM = 128
H1 = 64
H2 = 32

NCORE = 2
NSUB = 16
NT = NCORE * NSUB
EPT = E // NT
B = 128
NCHUNK = -(-EPT // B)
EPAD = NCHUNK * B - EPT
NPAD = 10240
PADN = NPAD - 1
RPS = NPAD // NSUB

_MESH = plsc.VectorSubcoreMesh(core_axis_name="c", subcore_axis_name="s")
_SC_PARAMS = pltpu.CompilerParams(use_tc_tiling_on_sc=False, needs_layout_passes=False)
_f32 = jnp.float32


W0 = 16
NROW = NPAD * 4 // W0
RPS0 = NROW // NSUB
NCOMB = NROW // 128


@functools.partial(
    pl.kernel,
    out_type=jax.ShapeDtypeStruct((NCORE, NROW, W0), _f32),
    mesh=_MESH,
    compiler_params=_SC_PARAMS,
    scratch_types=[
        pltpu.VMEM((NCHUNK, B), jnp.int32),
        pltpu.VMEM((NCHUNK, B), jnp.int32),
        pltpu.VMEM((NCHUNK, B), _f32),
        pltpu.VMEM((NCOMB, 128), jnp.int32),
        pltpu.VMEM((NROW, W0), _f32),
        pltpu.VMEM_SHARED((NROW, W0), _f32),
    ],
)
def _s0(src_h, dst_h, w_h, z2_h, idr_h, p4_h,
        src_v, dst_v, w_v, idr_v, acc_v, sh):
    c = lax.axis_index("c")
    s = lax.axis_index("s")
    wid = c * NSUB + s
    pltpu.sync_copy(src_h.at[wid], src_v)
    pltpu.sync_copy(dst_h.at[wid], dst_v)
    pltpu.sync_copy(w_h.at[wid], w_v)
    pltpu.sync_copy(idr_h, idr_v)
    pltpu.sync_copy(z2_h.at[pl.ds(0, NROW)], acc_v)
    pltpu.sync_copy(z2_h.at[pl.ds(s * RPS0, RPS0)], sh.at[pl.ds(s * RPS0, RPS0)])

    ones = jnp.ones((16,), _f32)
    one_i = jnp.ones((16,), jnp.int32)

    def body(j, carry):
        for e0 in range(B // 16):
            sl = pl.ds(e0 * 16, 16)
            s16 = src_v[j, sl]
            d16 = dst_v[j, sl]
            w16 = w_v[j, sl]
            rs = lax.shift_right_logical(s16, 2)
            cs = lax.shift_left(s16 & 3, 2)
            rd = lax.shift_right_logical(d16, 2)
            cd = lax.shift_left(d16 & 3, 2) + 2
            plsc.addupdate_scatter(acc_v, [rs, cs], w16)
            plsc.addupdate_scatter(acc_v, [rs, cs + one_i], ones)
            plsc.addupdate_scatter(acc_v, [rd, cd], w16)
            plsc.addupdate_scatter(acc_v, [rd, cd + one_i], ones)
        return carry

    lax.fori_loop(0, NCHUNK, body, 0)
    plsc.subcore_barrier()
    for t in range(NCOMB):
        pltpu.sync_copy(acc_v.at[pl.ds(t * 128, 128)], sh.at[idr_v.at[t]], add=True)
    plsc.subcore_barrier()
    pltpu.sync_copy(sh.at[pl.ds(s * RPS0, RPS0)], p4_h.at[c, pl.ds(s * RPS0, RPS0)])


def _rsqrt16(x):
    xb = lax.bitcast_convert_type(x, jnp.int32)
    yb = jnp.int32(0x5F3759DF) - lax.shift_right_logical(xb, 1)
    y = lax.bitcast_convert_type(yb, _f32)
    for _ in range(3):
        y = y * (1.5 - 0.5 * x * y * y)
    return jnp.where(x > 0.0, y, 0.0)


@functools.partial(
    pl.kernel,
    out_type=jax.ShapeDtypeStruct((NCORE, NPAD, H1), _f32),
    mesh=_MESH,
    compiler_params=_SC_PARAMS,
    scratch_types=[
        pltpu.VMEM((NCHUNK, B), jnp.int32),
        pltpu.VMEM((NCHUNK, B), jnp.int32),
        pltpu.VMEM((NCHUNK, B), _f32),
        pltpu.VMEM((2, B, H1), _f32),
        pltpu.VMEM((RPS0, W0), _f32),
        pltpu.VMEM((RPS0, W0), _f32),
        pltpu.VMEM((RPS,), _f32),
        pltpu.VMEM((NPAD,), _f32),
        pltpu.VMEM_SHARED((NPAD,), _f32),
        pltpu.VMEM_SHARED((NPAD, H1), _f32),
        pltpu.SemaphoreType.DMA((2,)),
        pltpu.SemaphoreType.DMA((2,)),
    ],
)
def _s1(g_h, p4_h, src_h, dst_h, w_h, z_h, acc_h,
        src_v, dst_v, w_v, rows_v, pa_v, pb_v, snbuf, sn_v, sn_sh, sh_acc,
        gsem, ssem):
    c = lax.axis_index("c")
    s = lax.axis_index("s")
    wid = c * NSUB + s
    pltpu.sync_copy(src_h.at[wid], src_v)
    pltpu.sync_copy(dst_h.at[wid], dst_v)
    pltpu.sync_copy(w_h.at[wid], w_v)
    pltpu.sync_copy(z_h.at[pl.ds(s * RPS, RPS)], sh_acc.at[pl.ds(s * RPS, RPS)])

    pltpu.sync_copy(p4_h.at[0, pl.ds(s * RPS0, RPS0)], pa_v)
    pltpu.sync_copy(p4_h.at[1, pl.ds(s * RPS0, RPS0)], pb_v)
    iv16 = lax.iota(jnp.int32, 16)
    for t in range(RPS // 16):
        iv = iv16 + (t * 16)
        r = lax.shift_right_logical(iv, 2)
        cc = lax.shift_left(iv & 3, 2)
        ow = plsc.load_gather(pa_v, [r, cc]) + plsc.load_gather(pb_v, [r, cc])
        snbuf[pl.ds(t * 16, 16)] = _rsqrt16(ow)
    pltpu.sync_copy(snbuf, sn_sh.at[pl.ds(s * RPS, RPS)])
    plsc.subcore_barrier()
    pltpu.sync_copy(sn_sh, sn_v)

    pltpu.async_copy(g_h.at[src_v.at[0]], rows_v.at[0], gsem.at[0])

    def chunk(j, carry):
        cb = lax.rem(j, 2)
        nb = 1 - cb

        @pl.when(jnp.logical_and(j + 1 < NCHUNK, j >= 1))
        def _():
            pltpu.make_async_copy(
                rows_v.at[nb], sh_acc.at[dst_v.at[j - 1]], ssem.at[nb]).wait()

        @pl.when(j + 1 < NCHUNK)
        def _():
            pltpu.async_copy(g_h.at[src_v.at[j + 1]], rows_v.at[nb], gsem.at[nb])

        pltpu.make_async_copy(g_h.at[src_v.at[j]], rows_v.at[cb], gsem.at[cb]).wait()

        for e0 in range(B // 16):
            sl0 = pl.ds(e0 * 16, 16)
            s16 = src_v[j, sl0]
            wv = w_v[j, sl0] * plsc.load_gather(sn_v, [s16])
            for l in range(16):
                e = e0 * 16 + l
                we = wv[l]
                for k in range(H1 // 16):
                    sl = pl.ds(k * 16, 16)
                    rows_v[cb, e, sl] = rows_v[cb, e, sl] * we

        pltpu.async_copy(rows_v.at[cb], sh_acc.at[dst_v.at[j]], ssem.at[cb], add=True)
        return carry

    lax.fori_loop(0, NCHUNK, chunk, 0)
    for jt in (NCHUNK - 2, NCHUNK - 1):
        pltpu.make_async_copy(
            rows_v.at[jt % 2], sh_acc.at[dst_v.at[jt]], ssem.at[jt % 2]).wait()
    plsc.subcore_barrier()
    pltpu.sync_copy(sh_acc.at[pl.ds(s * RPS, RPS)], acc_h.at[c, pl.ds(s * RPS, RPS)])


@functools.partial(
    pl.kernel,
    out_type=jax.ShapeDtypeStruct((NCORE, NPAD, H2), _f32),
    mesh=_MESH,
    compiler_params=_SC_PARAMS,
    scratch_types=[
        pltpu.VMEM((NCHUNK, B), jnp.int32),
        pltpu.VMEM((NCHUNK, B), jnp.int32),
        pltpu.VMEM((2, B, H2), _f32),
        pltpu.VMEM_SHARED((NPAD, H2), _f32),
        pltpu.SemaphoreType.DMA((2,)),
        pltpu.SemaphoreType.DMA((2,)),
    ],
)
def _s2(h2_h, src_h, dst_h, z_h, agg_h, src_v, dst_v, rows_v, sh_acc, gsem, ssem):
    c = lax.axis_index("c")
    s = lax.axis_index("s")
    wid = c * NSUB + s
    pltpu.sync_copy(src_h.at[wid], src_v)
    pltpu.sync_copy(dst_h.at[wid], dst_v)
    pltpu.sync_copy(z_h.at[pl.ds(s * RPS, RPS)], sh_acc.at[pl.ds(s * RPS, RPS)])
    plsc.subcore_barrier()

    pltpu.async_copy(h2_h.at[src_v.at[0]], rows_v.at[0], gsem.at[0])

    def chunk(j, carry):
        cb = lax.rem(j, 2)
        nb = 1 - cb

        @pl.when(jnp.logical_and(j + 1 < NCHUNK, j >= 1))
        def _():
            pltpu.make_async_copy(
                rows_v.at[nb], sh_acc.at[dst_v.at[j - 1]], ssem.at[nb]).wait()

        @pl.when(j + 1 < NCHUNK)
        def _():
            pltpu.async_copy(h2_h.at[src_v.at[j + 1]], rows_v.at[nb], gsem.at[nb])

        pltpu.make_async_copy(h2_h.at[src_v.at[j]], rows_v.at[cb], gsem.at[cb]).wait()
        pltpu.async_copy(rows_v.at[cb], sh_acc.at[dst_v.at[j]], ssem.at[cb], add=True)
        return carry

    lax.fori_loop(0, NCHUNK, chunk, 0)
    for jt in (NCHUNK - 2, NCHUNK - 1):
        pltpu.make_async_copy(
            rows_v.at[jt % 2], sh_acc.at[dst_v.at[jt]], ssem.at[jt % 2]).wait()
    plsc.subcore_barrier()
    pltpu.sync_copy(sh_acc.at[pl.ds(s * RPS, RPS)], agg_h.at[c, pl.ds(s * RPS, RPS)])


def _k1_body(f_ref, w1_ref, g_ref):
    g_ref[...] = jnp.dot(f_ref[...], w1_ref[...], preferred_element_type=_f32)


def _k2_body(acc_ref, p4_ref, b1_ref, w2_ref, h2_ref):
    p4 = p4_ref[0] + p4_ref[1]
    acc = acc_ref[0][:N] + acc_ref[1][:N]
    inw = p4[:N, 2:3]
    dn = jnp.where(inw > 0.0, lax.rsqrt(inw), 0.0)
    x1 = jnp.maximum(acc * dn + b1_ref[...], 0.0)
    od = p4[:N, 1:2]
    odn = lax.rsqrt(jnp.maximum(od, 1.0))
    h2_ref[...] = jnp.dot(x1 * odn, w2_ref[...], preferred_element_type=_f32)


def _k3_body(agg_ref, p4_ref, b2_ref, out_ref):
    agg = agg_ref[0][:N] + agg_ref[1][:N]
    ind = (p4_ref[0] + p4_ref[1])[:N, 3:4]
    out_ref[...] = agg * lax.rsqrt(jnp.maximum(ind, 1.0)) + b2_ref[...]


def kernel(features, edge_index, edge_weight, W1, b1, W2, b2):
    pad = ((0, 0), (0, EPAD))
    src = jnp.pad(edge_index[0].astype(jnp.int32).reshape(NT, EPT), pad,
                  constant_values=PADN).reshape(NT, NCHUNK, B)
    dst = jnp.pad(edge_index[1].astype(jnp.int32).reshape(NT, EPT), pad,
                  constant_values=PADN).reshape(NT, NCHUNK, B)
    w3 = jnp.pad(edge_weight.astype(_f32).reshape(NT, EPT), pad,
                 constant_values=0.0).reshape(NT, NCHUNK, B)
    idrows = jnp.arange(NROW, dtype=jnp.int32).reshape(NCOMB, 128)
    z2 = jnp.zeros((NPAD, W0), _f32)
    z64 = jnp.zeros((NPAD, H1), _f32)
    z32 = jnp.zeros((NPAD, H2), _f32)

    p4w = _s0(src, dst, w3, z2, idrows)
    p4 = p4w.reshape(NCORE, NPAD, 4)

    g = pl.pallas_call(
        _k1_body,
        out_shape=jax.ShapeDtypeStruct((N, H1), _f32),
    )(features, W1)

    acc = _s1(g, p4w, src, dst, w3, z64)

    h2 = pl.pallas_call(
        _k2_body,
        out_shape=jax.ShapeDtypeStruct((N, H2), _f32),
    )(acc, p4, b1.reshape(1, H1), W2)

    agg = _s2(h2, src, dst, z32)

    x2 = pl.pallas_call(
        _k3_body,
        out_shape=jax.ShapeDtypeStruct((N, H2), _f32),
    )(agg, p4, b2.reshape(1, H2))
    return x2


# speedup vs baseline: 1.2668x; 1.2668x over previous
import functools

import jax
import jax.numpy as jnp
from jax import lax
from jax.experimental import pallas as pl
from jax.experimental.pallas import tpu as pltpu
from jax.experimental.pallas import tpu_sc as plsc

N = 10000
E = 320000
IN_DIM = 128
H1 = 64
H2 = 32

NCORE = 2
NSUB = 16
NT = NCORE * NSUB
EPT = E // NT
B = 80
NCHUNK = EPT // B
EPAD = NCHUNK * B - EPT
NPAD = 10240
PADN = NPAD - 1
RPS = NPAD // NSUB

_MESH = plsc.VectorSubcoreMesh(core_axis_name="c", subcore_axis_name="s")
_SC_PARAMS = pltpu.CompilerParams(use_tc_tiling_on_sc=False, needs_layout_passes=False)
_f32 = jnp.float32


W0 = 16
NROW = NPAD * 4 // W0
RPS0 = NROW // NSUB
NCOMB = NROW // 128


@functools.partial(
    pl.kernel,
    out_type=jax.ShapeDtypeStruct((NCORE, NROW, W0), _f32),
    mesh=_MESH,
    compiler_params=_SC_PARAMS,
    scratch_types=[
        pltpu.VMEM((NCHUNK, B), jnp.int32),
        pltpu.VMEM((NCHUNK, B), jnp.int32),
        pltpu.VMEM((NCHUNK, B), _f32),
        pltpu.VMEM((NCOMB, 128), jnp.int32),
        pltpu.VMEM((NROW, W0), _f32),
        pltpu.VMEM_SHARED((NROW, W0), _f32),
    ],
)
def _s0(src_h, dst_h, w_h, z2_h, idr_h, p4_h,
        src_v, dst_v, w_v, idr_v, acc_v, sh):
    c = lax.axis_index("c")
    s = lax.axis_index("s")
    wid = c * NSUB + s
    pltpu.sync_copy(src_h.at[wid], src_v)
    pltpu.sync_copy(dst_h.at[wid], dst_v)
    pltpu.sync_copy(w_h.at[wid], w_v)
    pltpu.sync_copy(idr_h, idr_v)
    pltpu.sync_copy(z2_h.at[pl.ds(0, NROW)], acc_v)
    pltpu.sync_copy(z2_h.at[pl.ds(s * RPS0, RPS0)], sh.at[pl.ds(s * RPS0, RPS0)])

    ones = jnp.ones((16,), _f32)
    one_i = jnp.ones((16,), jnp.int32)

    def body(j, carry):
        for e0 in range(B // 16):
            sl = pl.ds(e0 * 16, 16)
            s16 = src_v[j, sl]
            d16 = dst_v[j, sl]
            w16 = w_v[j, sl]
            rs = lax.shift_right_logical(s16, 2)
            cs = lax.shift_left(s16 & 3, 2)
            rd = lax.shift_right_logical(d16, 2)
            cd = lax.shift_left(d16 & 3, 2) + 2
            plsc.addupdate_scatter(acc_v, [rs, cs], w16)
            plsc.addupdate_scatter(acc_v, [rs, cs + one_i], ones)
            plsc.addupdate_scatter(acc_v, [rd, cd], w16)
            plsc.addupdate_scatter(acc_v, [rd, cd + one_i], ones)
        return carry

    lax.fori_loop(0, NCHUNK, body, 0)
    plsc.subcore_barrier()
    for t in range(NCOMB):
        pltpu.sync_copy(acc_v.at[pl.ds(t * 128, 128)], sh.at[idr_v.at[t]], add=True)
    plsc.subcore_barrier()
    pltpu.sync_copy(sh.at[pl.ds(s * RPS0, RPS0)], p4_h.at[c, pl.ds(s * RPS0, RPS0)])


def _rsqrt16(x):
    xb = lax.bitcast_convert_type(x, jnp.int32)
    yb = jnp.int32(0x5F3759DF) - lax.shift_right_logical(xb, 1)
    y = lax.bitcast_convert_type(yb, _f32)
    for _ in range(3):
        y = y * (1.5 - 0.5 * x * y * y)
    return jnp.where(x > 0.0, y, 0.0)


@functools.partial(
    pl.kernel,
    out_type=jax.ShapeDtypeStruct((NCORE, NPAD, H1), _f32),
    mesh=_MESH,
    compiler_params=_SC_PARAMS,
    scratch_types=[
        pltpu.VMEM((NCHUNK, B), jnp.int32),
        pltpu.VMEM((NCHUNK, B), jnp.int32),
        pltpu.VMEM((NCHUNK, B), _f32),
        pltpu.VMEM((2, B, H1), _f32),
        pltpu.VMEM((RPS0, W0), _f32),
        pltpu.VMEM((RPS0, W0), _f32),
        pltpu.VMEM((RPS,), _f32),
        pltpu.VMEM((NPAD,), _f32),
        pltpu.VMEM_SHARED((NPAD,), _f32),
        pltpu.VMEM_SHARED((NPAD, H1), _f32),
        pltpu.SemaphoreType.DMA((2,)),
        pltpu.SemaphoreType.DMA((2,)),
    ],
)
def _s1(g_h, p4_h, src_h, dst_h, w_h, z_h, acc_h,
        src_v, dst_v, w_v, rows_v, pa_v, pb_v, snbuf, sn_v, sn_sh, sh_acc,
        gsem, ssem):
    c = lax.axis_index("c")
    s = lax.axis_index("s")
    wid = c * NSUB + s
    pltpu.sync_copy(src_h.at[wid], src_v)
    pltpu.sync_copy(dst_h.at[wid], dst_v)
    pltpu.sync_copy(w_h.at[wid], w_v)
    pltpu.sync_copy(z_h.at[pl.ds(s * RPS, RPS)], sh_acc.at[pl.ds(s * RPS, RPS)])

    pltpu.sync_copy(p4_h.at[0, pl.ds(s * RPS0, RPS0)], pa_v)
    pltpu.sync_copy(p4_h.at[1, pl.ds(s * RPS0, RPS0)], pb_v)
    iv16 = lax.iota(jnp.int32, 16)
    for t in range(RPS // 16):
        iv = iv16 + (t * 16)
        r = lax.shift_right_logical(iv, 2)
        cc = lax.shift_left(iv & 3, 2)
        ow = plsc.load_gather(pa_v, [r, cc]) + plsc.load_gather(pb_v, [r, cc])
        snbuf[pl.ds(t * 16, 16)] = _rsqrt16(ow)
    pltpu.sync_copy(snbuf, sn_sh.at[pl.ds(s * RPS, RPS)])
    plsc.subcore_barrier()
    pltpu.sync_copy(sn_sh, sn_v)

    pltpu.async_copy(g_h.at[src_v.at[0]], rows_v.at[0], gsem.at[0])

    def chunk(j, carry):
        cb = lax.rem(j, 2)
        nb = 1 - cb

        @pl.when(jnp.logical_and(j + 1 < NCHUNK, j >= 1))
        def _():
            pltpu.make_async_copy(
                rows_v.at[nb], sh_acc.at[dst_v.at[j - 1]], ssem.at[nb]).wait()

        @pl.when(j + 1 < NCHUNK)
        def _():
            pltpu.async_copy(g_h.at[src_v.at[j + 1]], rows_v.at[nb], gsem.at[nb])

        pltpu.make_async_copy(g_h.at[src_v.at[j]], rows_v.at[cb], gsem.at[cb]).wait()

        for e0 in range(B // 16):
            sl0 = pl.ds(e0 * 16, 16)
            s16 = src_v[j, sl0]
            wv = w_v[j, sl0] * plsc.load_gather(sn_v, [s16])
            for l in range(16):
                e = e0 * 16 + l
                we = wv[l]
                for k in range(H1 // 16):
                    sl = pl.ds(k * 16, 16)
                    rows_v[cb, e, sl] = rows_v[cb, e, sl] * we

        pltpu.async_copy(rows_v.at[cb], sh_acc.at[dst_v.at[j]], ssem.at[cb], add=True)
        return carry

    lax.fori_loop(0, NCHUNK, chunk, 0)
    for jt in (NCHUNK - 2, NCHUNK - 1):
        pltpu.make_async_copy(
            rows_v.at[jt % 2], sh_acc.at[dst_v.at[jt]], ssem.at[jt % 2]).wait()
    plsc.subcore_barrier()
    pltpu.sync_copy(sh_acc.at[pl.ds(s * RPS, RPS)], acc_h.at[c, pl.ds(s * RPS, RPS)])


@functools.partial(
    pl.kernel,
    out_type=jax.ShapeDtypeStruct((NCORE, NPAD, H2), _f32),
    mesh=_MESH,
    compiler_params=_SC_PARAMS,
    scratch_types=[
        pltpu.VMEM((NCHUNK, B), jnp.int32),
        pltpu.VMEM((NCHUNK, B), jnp.int32),
        pltpu.VMEM((2, B, H2), _f32),
        pltpu.VMEM_SHARED((NPAD, H2), _f32),
        pltpu.SemaphoreType.DMA((2,)),
        pltpu.SemaphoreType.DMA((2,)),
    ],
)
def _s2(h2_h, src_h, dst_h, z_h, agg_h, src_v, dst_v, rows_v, sh_acc, gsem, ssem):
    c = lax.axis_index("c")
    s = lax.axis_index("s")
    wid = c * NSUB + s
    pltpu.sync_copy(src_h.at[wid], src_v)
    pltpu.sync_copy(dst_h.at[wid], dst_v)
    pltpu.sync_copy(z_h.at[pl.ds(s * RPS, RPS)], sh_acc.at[pl.ds(s * RPS, RPS)])
    plsc.subcore_barrier()

    pltpu.async_copy(h2_h.at[src_v.at[0]], rows_v.at[0], gsem.at[0])

    def chunk(j, carry):
        cb = lax.rem(j, 2)
        nb = 1 - cb

        @pl.when(jnp.logical_and(j + 1 < NCHUNK, j >= 1))
        def _():
            pltpu.make_async_copy(
                rows_v.at[nb], sh_acc.at[dst_v.at[j - 1]], ssem.at[nb]).wait()

        @pl.when(j + 1 < NCHUNK)
        def _():
            pltpu.async_copy(h2_h.at[src_v.at[j + 1]], rows_v.at[nb], gsem.at[nb])

        pltpu.make_async_copy(h2_h.at[src_v.at[j]], rows_v.at[cb], gsem.at[cb]).wait()
        pltpu.async_copy(rows_v.at[cb], sh_acc.at[dst_v.at[j]], ssem.at[cb], add=True)
        return carry

    lax.fori_loop(0, NCHUNK, chunk, 0)
    for jt in (NCHUNK - 2, NCHUNK - 1):
        pltpu.make_async_copy(
            rows_v.at[jt % 2], sh_acc.at[dst_v.at[jt]], ssem.at[jt % 2]).wait()
    plsc.subcore_barrier()
    pltpu.sync_copy(sh_acc.at[pl.ds(s * RPS, RPS)], agg_h.at[c, pl.ds(s * RPS, RPS)])


def _k1_body(f_ref, w1_ref, g_ref):
    g_ref[...] = jnp.dot(f_ref[...], w1_ref[...], preferred_element_type=_f32)


def _k2_body(acc_ref, p4_ref, b1_ref, w2_ref, h2_ref):
    p4 = p4_ref[0] + p4_ref[1]
    acc = acc_ref[0][:N] + acc_ref[1][:N]
    inw = p4[:N, 2:3]
    dn = jnp.where(inw > 0.0, lax.rsqrt(inw), 0.0)
    x1 = jnp.maximum(acc * dn + b1_ref[...], 0.0)
    od = p4[:N, 1:2]
    odn = lax.rsqrt(jnp.maximum(od, 1.0))
    h2_ref[...] = jnp.dot(x1 * odn, w2_ref[...], preferred_element_type=_f32)


def _k3_body(agg_ref, p4_ref, b2_ref, out_ref):
    agg = agg_ref[0][:N] + agg_ref[1][:N]
    ind = (p4_ref[0] + p4_ref[1])[:N, 3:4]
    out_ref[...] = agg * lax.rsqrt(jnp.maximum(ind, 1.0)) + b2_ref[...]


def kernel(features, edge_index, edge_weight, W1, b1, W2, b2):
    src = edge_index[0].astype(jnp.int32).reshape(NT, NCHUNK, B)
    dst = edge_index[1].astype(jnp.int32).reshape(NT, NCHUNK, B)
    w3 = edge_weight.astype(_f32).reshape(NT, NCHUNK, B)
    idrows = jnp.arange(NROW, dtype=jnp.int32).reshape(NCOMB, 128)
    z2 = jnp.zeros((NPAD, W0), _f32)
    z64 = jnp.zeros((NPAD, H1), _f32)
    z32 = jnp.zeros((NPAD, H2), _f32)

    p4w = _s0(src, dst, w3, z2, idrows)
    p4 = p4w.reshape(NCORE, NPAD, 4)

    g = pl.pallas_call(
        _k1_body,
        out_shape=jax.ShapeDtypeStruct((N, H1), _f32),
    )(features, W1)

    acc = _s1(g, p4w, src, dst, w3, z64)

    h2 = pl.pallas_call(
        _k2_body,
        out_shape=jax.ShapeDtypeStruct((N, H2), _f32),
    )(acc, p4, b1.reshape(1, H1), W2)

    agg = _s2(h2, src, dst, z32)

    x2 = pl.pallas_call(
        _k3_body,
        out_shape=jax.ShapeDtypeStruct((N, H2), _f32),
    )(agg, p4, b2.reshape(1, H2))
    return x2
